# baseline (speedup 1.0000x reference)
import jax
import jax.numpy as jnp
from jax.experimental import pallas as pl
from jax.experimental.pallas import tpu as pltpu

NUM_LABELS = 1000
NUM_WEIGHTS = 256
EMB_DIM = 64
BATCH = 4096
IMG_DIM = 1024
OUT_DIM = EMB_DIM + IMG_DIM
TOTAL_ROWS = NUM_WEIGHTS + BATCH
TABLE_PAD = NUM_LABELS + 1

CHUNK = 1024
NCHUNK = BATCH // CHUNK
SPLIT = 2


def _split_copies(src, dst, sems, nrows):
    step = nrows // SPLIT
    cps = []
    for s in range(SPLIT):
        cps.append(pltpu.make_async_copy(
            src.at[pl.ds(s * step, step)], dst.at[pl.ds(s * step, step)],
            sems[s]))
    return cps


def _tc_body(lbl_hbm, table_ref, w_hbm, img_hbm, out_hbm,
             ib0, ib1, ob0, ob1, tb, wv, lblv,
             isems0, isems1, osems0, osems1, tsem, wsem, lsem):
    ibufs, obufs = [ib0, ib1], [ob0, ob1]
    isems, osems = [isems0, isems1], [osems0, osems1]

    def start_in(i, sl):
        cps = _split_copies(
            img_hbm.at[pl.ds(i * CHUNK, CHUNK)], ibufs[sl], isems[sl], CHUNK)
        for c in cps:
            c.start()
        return cps

    wcp = pltpu.make_async_copy(w_hbm, wv, wsem)
    wcp.start()
    lcp = pltpu.make_async_copy(lbl_hbm, lblv, lsem)
    lcp.start()

    started_in = [start_in(0, 0), start_in(1, 1)]

    wcp.wait()
    tb[...] = jnp.concatenate(
        [wv[...], jnp.zeros((NUM_WEIGHTS, IMG_DIM), jnp.float32)], axis=1)
    tcp = pltpu.make_async_copy(tb, out_hbm.at[pl.ds(0, NUM_WEIGHTS)], tsem)
    tcp.start()
    lcp.wait()

    started_out = {}
    for i in range(NCHUNK):
        sl = i % 2
        lbl = lblv[pl.ds(i * CHUNK, CHUNK), :]
        iota = jax.lax.broadcasted_iota(jnp.int32, (CHUNK, TABLE_PAD), 1)
        onehot = (iota == lbl).astype(jnp.float32)
        enc = jax.lax.dot_general(
            onehot, table_ref[...],
            dimension_numbers=(((1,), (0,)), ((), ())),
            preferred_element_type=jnp.float32,
        )
        for c in started_in[i]:
            c.wait()
        if i >= 2:
            for c in started_out[i - 2]:
                c.wait()
        obufs[sl][...] = jnp.concatenate([enc, ibufs[sl][...]], axis=1)
        ocps = _split_copies(
            obufs[sl], out_hbm.at[pl.ds(NUM_WEIGHTS + i * CHUNK, CHUNK)],
            osems[sl], CHUNK)
        for c in ocps:
            c.start()
        started_out[i] = ocps
        if i + 2 < NCHUNK:
            started_in.append(start_in(i + 2, sl))

    for c in started_out[NCHUNK - 2]:
        c.wait()
    for c in started_out[NCHUNK - 1]:
        c.wait()
    tcp.wait()


@jax.jit
def kernel(images, labels, label_embs, weight_embs):
    lbl2d = labels.reshape(BATCH, 1)

    out = pl.pallas_call(
        _tc_body,
        in_specs=[
            pl.BlockSpec(memory_space=pl.ANY),
            pl.BlockSpec(memory_space=pltpu.VMEM),
            pl.BlockSpec(memory_space=pl.ANY),
            pl.BlockSpec(memory_space=pl.ANY),
        ],
        out_specs=pl.BlockSpec(memory_space=pl.ANY),
        out_shape=jax.ShapeDtypeStruct((TOTAL_ROWS, OUT_DIM), jnp.float32),
        scratch_shapes=[
            pltpu.VMEM((CHUNK, IMG_DIM), jnp.float32),
            pltpu.VMEM((CHUNK, IMG_DIM), jnp.float32),
            pltpu.VMEM((CHUNK, OUT_DIM), jnp.float32),
            pltpu.VMEM((CHUNK, OUT_DIM), jnp.float32),
            pltpu.VMEM((NUM_WEIGHTS, OUT_DIM), jnp.float32),
            pltpu.VMEM((NUM_WEIGHTS, EMB_DIM), jnp.float32),
            pltpu.VMEM((BATCH, 1), jnp.int32),
            [pltpu.SemaphoreType.DMA] * SPLIT,
            [pltpu.SemaphoreType.DMA] * SPLIT,
            [pltpu.SemaphoreType.DMA] * SPLIT,
            [pltpu.SemaphoreType.DMA] * SPLIT,
            pltpu.SemaphoreType.DMA,
            pltpu.SemaphoreType.DMA,
            pltpu.SemaphoreType.DMA,
        ],
        compiler_params=pltpu.CompilerParams(
            vmem_limit_bytes=100 * 1024 * 1024,
        ),
    )(lbl2d, label_embs, weight_embs, images)
    return out
